# Initial kernel scaffold; baseline (speedup 1.0000x reference)
import jax
import jax.numpy as jnp
from jax.experimental import pallas as pl


def kernel(x, table, W, b):
    raise NotImplementedError("write your pallas kernel here")


# speedup vs baseline: 2.4861x; 2.4861x over previous
import functools

import jax
import jax.numpy as jnp
from jax import lax
from jax.experimental import pallas as pl
from jax.experimental.pallas import tpu as pltpu
from jax.experimental.pallas import tpu_sc as plsc

VOCAB = 1000000
D = 64
NCLS = 100
B = 16384
H = 50

NC, NS = 2, 16
NW = NC * NS
SPW = B // NW
CH = 16
NCHUNK = SPW // CH
RPC = CH * H


def _sc_pool_sums(table, xf):
  mesh = plsc.VectorSubcoreMesh(
      core_axis_name="c", subcore_axis_name="s", num_cores=NC, num_subcores=NS
  )

  @functools.partial(
      pl.kernel,
      out_type=jax.ShapeDtypeStruct((B, D), jnp.float32),
      mesh=mesh,
      scratch_types=[
          pltpu.VMEM((CH, H), jnp.int32),
          pltpu.VMEM((RPC, D), jnp.float32),
          pltpu.VMEM((CH, D), jnp.float32),
          pltpu.SemaphoreType.DMA,
      ],
      compiler_params=pltpu.CompilerParams(use_tc_tiling_on_sc=False),
  )
  def k(table_hbm, xf_hbm, out_hbm, idx_v, rows_v, pool_v, sem):
    wid = lax.axis_index("s") * NC + lax.axis_index("c")

    @pl.loop(0, NCHUNK)
    def _chunk(c):
      sbase = wid * SPW + c * CH
      pltpu.sync_copy(xf_hbm.at[pl.ds(sbase, CH)], idx_v)
      cps = [
          pltpu.async_copy(
              table_hbm.at[idx_v.at[j]], rows_v.at[pl.ds(j * H, H)], sem
          )
          for j in range(CH)
      ]
      for cp in cps:
        cp.wait()

      @pl.loop(0, CH)
      def _sample(s):
        base = s * H

        def body(l, accs):
          r = base + l * 5
          out = accs
          for u in range(5):
            out = tuple(
                out[v] + rows_v[r + u, pl.ds(v * 16, 16)] for v in range(4)
            )
          return out

        accs = lax.fori_loop(
            0, H // 5, body,
            tuple(jnp.zeros((16,), jnp.float32) for _ in range(4)),
        )
        for v in range(4):
          pool_v[s, pl.ds(v * 16, 16)] = accs[v]

      pltpu.sync_copy(pool_v, out_hbm.at[pl.ds(sbase, CH)])

  return k(table, xf)


def _tc_linear(pooled, wt_pad, b_pad):
  bm = 2048

  def body(p_ref, wt_ref, b_ref, o_ref):
    acc = jnp.dot(p_ref[...], wt_ref[...], preferred_element_type=jnp.float32)
    o_ref[...] = acc * (1.0 / H) + b_ref[...]

  return pl.pallas_call(
      body,
      grid=(B // bm,),
      in_specs=[
          pl.BlockSpec((bm, D), lambda i: (i, 0)),
          pl.BlockSpec((D, 128), lambda i: (0, 0)),
          pl.BlockSpec((1, 128), lambda i: (0, 0)),
      ],
      out_specs=pl.BlockSpec((bm, 128), lambda i: (i, 0)),
      out_shape=jax.ShapeDtypeStruct((B, 128), jnp.float32),
  )(pooled, wt_pad, b_pad)


def kernel(x, table, W, b):
  xf = x.astype(jnp.int32)
  pooled = _sc_pool_sums(table, xf)
  wt_pad = jnp.zeros((D, 128), jnp.float32).at[:, :NCLS].set(W.T)
  b_pad = jnp.zeros((1, 128), jnp.float32).at[:, :NCLS].set(b.reshape(1, -1))
  out = _tc_linear(pooled, wt_pad, b_pad)
  return out[:, :NCLS]


# speedup vs baseline: 2.7412x; 1.1026x over previous
import functools

import jax
import jax.numpy as jnp
from jax import lax
from jax.experimental import pallas as pl
from jax.experimental.pallas import tpu as pltpu
from jax.experimental.pallas import tpu_sc as plsc

VOCAB = 1000000
D = 64
NCLS = 100
B = 16384
H = 50

NC, NS = 2, 16
NW = NC * NS
SPW = B // NW
CH = 16
NCHUNK = SPW // CH
RPC = CH * H


def _sc_pool_sums(table, xf):
  mesh = plsc.VectorSubcoreMesh(
      core_axis_name="c", subcore_axis_name="s", num_cores=NC, num_subcores=NS
  )

  @functools.partial(
      pl.kernel,
      out_type=jax.ShapeDtypeStruct((B, D), jnp.float32),
      mesh=mesh,
      scratch_types=[
          pltpu.VMEM((2, CH, H), jnp.int32),
          pltpu.VMEM((2, RPC, D), jnp.float32),
          pltpu.VMEM((CH, D), jnp.float32),
          pltpu.SemaphoreType.DMA,
          pltpu.SemaphoreType.DMA,
      ],
      compiler_params=pltpu.CompilerParams(use_tc_tiling_on_sc=False),
  )
  def k(table_hbm, xf_hbm, out_hbm, idx_v, rows_v, pool_v, sem0, sem1):
    wid = lax.axis_index("s") * NC + lax.axis_index("c")
    sems = (sem0, sem1)

    def load(cc, slot):
      sbase = wid * SPW + cc * CH
      pltpu.sync_copy(xf_hbm.at[pl.ds(sbase, CH)], idx_v.at[slot])
      for j in range(CH):
        pltpu.async_copy(
            table_hbm.at[idx_v.at[slot].at[j]],
            rows_v.at[slot].at[pl.ds(j * H, H)],
            sems[slot],
        )

    def drain(slot):
      pltpu.make_async_copy(
          table_hbm.at[pl.ds(0, RPC)], rows_v.at[slot], sems[slot]
      ).wait()

    def pool_store(cc, slot):
      @pl.loop(0, CH)
      def _sample(s):
        base = s * H

        def body(l, accs):
          r = base + l * 5
          out = accs
          for u in range(5):
            out = tuple(
                out[v] + rows_v[slot, r + u, pl.ds(v * 16, 16)]
                for v in range(4)
            )
          return out

        accs = lax.fori_loop(
            0, H // 5, body,
            tuple(jnp.zeros((16,), jnp.float32) for _ in range(4)),
        )
        for v in range(4):
          pool_v[s, pl.ds(v * 16, 16)] = accs[v]

      sbase = wid * SPW + cc * CH
      pltpu.sync_copy(pool_v, out_hbm.at[pl.ds(sbase, CH)])

    load(0, 0)

    @pl.loop(0, NCHUNK, step=2)
    def _chunk(c):
      for b in range(2):
        cc = c + b

        @pl.when(cc + 1 < NCHUNK)
        def _():
          load(cc + 1, (b + 1) % 2)

        drain(b)
        pool_store(cc, b)

  return k(table, xf)


def _tc_linear(pooled, wt_pad, b_pad):
  bm = 2048

  def body(p_ref, wt_ref, b_ref, o_ref):
    acc = jnp.dot(p_ref[...], wt_ref[...], preferred_element_type=jnp.float32)
    o_ref[...] = acc * (1.0 / H) + b_ref[...]

  return pl.pallas_call(
      body,
      grid=(B // bm,),
      in_specs=[
          pl.BlockSpec((bm, D), lambda i: (i, 0)),
          pl.BlockSpec((D, 128), lambda i: (0, 0)),
          pl.BlockSpec((1, 128), lambda i: (0, 0)),
      ],
      out_specs=pl.BlockSpec((bm, 128), lambda i: (i, 0)),
      out_shape=jax.ShapeDtypeStruct((B, 128), jnp.float32),
  )(pooled, wt_pad, b_pad)


def kernel(x, table, W, b):
  xf = x.astype(jnp.int32)
  pooled = _sc_pool_sums(table, xf)
  wt_pad = jnp.zeros((D, 128), jnp.float32).at[:, :NCLS].set(W.T)
  b_pad = jnp.zeros((1, 128), jnp.float32).at[:, :NCLS].set(b.reshape(1, -1))
  out = _tc_linear(pooled, wt_pad, b_pad)
  return out[:, :NCLS]
